# baseline (speedup 1.0000x reference)
import functools

import jax
import jax.numpy as jnp
from jax import lax
from jax.experimental import pallas as pl
from jax.experimental.pallas import tpu as pltpu
from jax.experimental.pallas import tpu_sc as plsc

N = 10000
D = 128
E = 320000

NC = 2
NS = 16
NW = NC * NS

K = 128
CH = 80
E_PAD = NW * CH * K
CHUNKS_REAL = E // K
ROWS_PER_SUB = 632
ROWS_LAST = N - (NS - 1) * ROWS_PER_SUB


def _sc_agg(x, src2, dst2):
    mesh = plsc.VectorSubcoreMesh(
        core_axis_name="c", subcore_axis_name="s",
        num_cores=NC, num_subcores=NS)

    HCH = CH // 2

    @functools.partial(
        pl.kernel,
        out_type=jax.ShapeDtypeStruct((NC, N, D), jnp.float32),
        mesh=mesh,
        scratch_types=[
            pltpu.VMEM((HCH, K), jnp.int32),
            pltpu.VMEM((HCH, K), jnp.int32),
            pltpu.VMEM((K, D), jnp.float32),
            pltpu.VMEM((K, D), jnp.float32),
            pltpu.VMEM_SHARED((N, D), jnp.float32),
            pltpu.SemaphoreType.DMA,
            pltpu.SemaphoreType.DMA,
            pltpu.SemaphoreType.DMA,
            pltpu.SemaphoreType.DMA,
            pltpu.SemaphoreType.DMA,
            pltpu.SemaphoreType.DMA,
        ],
    )
    def agg_kernel(x_hbm, src_hbm, dst_hbm, out_hbm,
                   sidx, didx, rows0, rows1, acc,
                   g0a, g0b, g1a, g1b, ss0, ss1):
        c = lax.axis_index("c")
        s = lax.axis_index("s")
        wid = s * NC + c
        rc = jnp.minimum(CH, CHUNKS_REAL - wid * CH)

        pltpu.async_copy(src_hbm.at[pl.ds(wid * CH, HCH)], sidx, ss0)
        pltpu.async_copy(dst_hbm.at[pl.ds(wid * CH, HCH)], didx, ss1)

        r0 = s * ROWS_PER_SUB

        @pl.when(s < NS - 1)
        def _():
            pltpu.sync_copy(x_hbm.at[pl.ds(r0, ROWS_PER_SUB)],
                            acc.at[pl.ds(r0, ROWS_PER_SUB)])

        @pl.when(s == NS - 1)
        def _():
            pltpu.sync_copy(x_hbm.at[pl.ds((NS - 1) * ROWS_PER_SUB, ROWS_LAST)],
                            acc.at[pl.ds((NS - 1) * ROWS_PER_SUB, ROWS_LAST)])

        pltpu.make_async_copy(
            src_hbm.at[pl.ds(wid * CH, HCH)], sidx, ss0).wait()
        pltpu.make_async_copy(
            dst_hbm.at[pl.ds(wid * CH, HCH)], didx, ss1).wait()

        NQG = 1
        HK = K // NQG
        bufs = (rows0, rows1)
        gsems = ((g0a, g0b), (g1a, g1b))
        ssems = (ss0, ss1)

        def issue_gathers(loc, buf, sems):
            for q in range(NQG):
                pltpu.async_copy(
                    x_hbm.at[sidx.at[loc, pl.ds(q * HK, HK)]],
                    buf.at[pl.ds(q * HK, HK)], sems[q])

        def wait_gathers(loc, buf, sems):
            for q in range(NQG):
                pltpu.make_async_copy(
                    x_hbm.at[sidx.at[loc, pl.ds(q * HK, HK)]],
                    buf.at[pl.ds(q * HK, HK)], sems[q]).wait()

        issue_gathers(0, rows0, gsems[0])
        plsc.subcore_barrier()

        for h in range(2):
            base = h * HCH

            if h == 1:
                pltpu.sync_copy(src_hbm.at[pl.ds(wid * CH + base, HCH)], sidx)
                pltpu.sync_copy(dst_hbm.at[pl.ds(wid * CH + base, HCH)], didx)

                @pl.when(base < rc)
                def _():
                    issue_gathers(0, rows0, gsems[0])

            def body(j, carry):
                for b in range(2):
                    loc = j * 2 + b
                    ch = base + loc
                    buf, gsem, ssem = bufs[b], gsems[b], ssems[b]
                    nbuf, ngsem, nssem = bufs[1 - b], gsems[1 - b], ssems[1 - b]

                    @pl.when(ch < rc)
                    def _():
                        @pl.when(jnp.logical_and(loc + 1 < HCH, ch + 1 < rc))
                        def _():
                            @pl.when(loc >= 1)
                            def _():
                                pltpu.make_async_copy(
                                    nbuf, acc.at[didx.at[0]], nssem).wait()

                            issue_gathers(loc + 1, nbuf, ngsem)

                        wait_gathers(loc, buf, gsem)
                        pltpu.async_copy(
                            buf, acc.at[didx.at[loc]], ssem, add=True)

                return carry

            lax.fori_loop(0, HCH // 2, body, 0)

            @pl.when(base < rc)
            def _():
                pltpu.make_async_copy(rows0, acc.at[didx.at[0]], ss0).wait()
                pltpu.make_async_copy(rows1, acc.at[didx.at[0]], ss1).wait()

        plsc.subcore_barrier()

        @pl.when(s < NS - 1)
        def _():
            pltpu.sync_copy(acc.at[pl.ds(r0, ROWS_PER_SUB)],
                            out_hbm.at[c, pl.ds(r0, ROWS_PER_SUB)])

        @pl.when(s == NS - 1)
        def _():
            pltpu.sync_copy(acc.at[pl.ds((NS - 1) * ROWS_PER_SUB, ROWS_LAST)],
                            out_hbm.at[c, pl.ds((NS - 1) * ROWS_PER_SUB, ROWS_LAST)])

    return agg_kernel(x, src2, dst2)


RB = 1000
NB = N // RB


def _tc_mlp(x, p, epsm1, W1, W2, g, b):
    def body(x_ref, p_ref, e_ref, w1_ref, w2_ref, g_ref, b_ref, o_ref,
             h_scr, st_scr):
        i = pl.program_id(0)
        h = x_ref[...] * e_ref[0, 0] + p_ref[0] + p_ref[1]
        h = lax.dot_general(h, w1_ref[...], (((1,), (1,)), ((), ())),
                            preferred_element_type=jnp.float32)
        h = jnp.maximum(h, 0.0)
        h = lax.dot_general(h, w2_ref[...], (((1,), (1,)), ((), ())),
                            preferred_element_type=jnp.float32)
        h = jnp.maximum(h, 0.0)
        h_scr[pl.ds(i * RB, RB), :] = h
        st = jnp.concatenate(
            [jnp.sum(h, axis=0, keepdims=True),
             jnp.sum(jnp.square(h), axis=0, keepdims=True)], axis=0)

        @pl.when(i == 0)
        def _():
            st_scr[...] = st

        @pl.when(i > 0)
        def _():
            st_scr[...] += st

        @pl.when(i == NB - 1)
        def _():
            mean = st_scr[0:1, :] * (1.0 / N)
            var = st_scr[1:2, :] * (1.0 / N) - jnp.square(mean)
            a = g_ref[...] * lax.rsqrt(var + 1e-5)
            cc = b_ref[...] - mean * a
            o_ref[...] = h_scr[...] * a + cc

    return pl.pallas_call(
        body,
        grid=(NB,),
        in_specs=[
            pl.BlockSpec((RB, D), lambda i: (i, 0)),
            pl.BlockSpec((NC, RB, D), lambda i: (0, i, 0)),
            pl.BlockSpec((1, 1), lambda i: (0, 0)),
            pl.BlockSpec((D, D), lambda i: (0, 0)),
            pl.BlockSpec((D, D), lambda i: (0, 0)),
            pl.BlockSpec((1, D), lambda i: (0, 0)),
            pl.BlockSpec((1, D), lambda i: (0, 0)),
        ],
        out_specs=pl.BlockSpec((N, D), lambda i: (0, 0)),
        out_shape=jax.ShapeDtypeStruct((N, D), jnp.float32),
        scratch_shapes=[
            pltpu.VMEM((N, D), jnp.float32),
            pltpu.VMEM((2, D), jnp.float32),
        ],
    )(x, p, epsm1, W1, W2, g.reshape(1, D), b.reshape(1, D))


def kernel(x, adj, eps0, W1_0, W2_0, g0, b0, eps1, W1_1, W2_1, g1, b1):
    pad = E_PAD - E
    src2 = jnp.pad(adj[0], (0, pad)).reshape(NW * CH, K)
    dst2 = jnp.pad(adj[1], (0, pad)).reshape(NW * CH, K)
    p = _sc_agg(x, src2, dst2)
    h = _tc_mlp(x, p, (eps0 - 1.0).reshape(1, 1), W1_0, W2_0, g0, b0)
    p2 = _sc_agg(h, src2, dst2)
    return _tc_mlp(h, p2, (eps1 - 1.0).reshape(1, 1), W1_1, W2_1, g1, b1)


# speedup vs baseline: 1.0214x; 1.0214x over previous
import functools

import jax
import jax.numpy as jnp
from jax import lax
from jax.experimental import pallas as pl
from jax.experimental.pallas import tpu as pltpu
from jax.experimental.pallas import tpu_sc as plsc

N = 10000
D = 128
E = 320000

NC = 2
NS = 16
NW = NC * NS

K = 128
CH = 80
E_PAD = NW * CH * K
CHUNKS_REAL = E // K
ROWS_PER_SUB = 632
ROWS_LAST = N - (NS - 1) * ROWS_PER_SUB


def _sc_agg(x, src2, dst2):
    mesh = plsc.VectorSubcoreMesh(
        core_axis_name="c", subcore_axis_name="s",
        num_cores=NC, num_subcores=NS)

    QCH = CH // 5

    @functools.partial(
        pl.kernel,
        out_type=jax.ShapeDtypeStruct((NC, N, D), jnp.float32),
        mesh=mesh,
        scratch_types=[
            pltpu.VMEM((CH, K), jnp.int32),
            pltpu.VMEM((QCH, K), jnp.int32),
            pltpu.VMEM((QCH, K), jnp.int32),
            pltpu.VMEM((K, D), jnp.float32),
            pltpu.VMEM((K, D), jnp.float32),
            pltpu.VMEM_SHARED((N, D), jnp.float32),
            pltpu.SemaphoreType.DMA,
            pltpu.SemaphoreType.DMA,
            pltpu.SemaphoreType.DMA,
            pltpu.SemaphoreType.DMA,
            pltpu.SemaphoreType.DMA,
            pltpu.SemaphoreType.DMA,
        ],
    )
    def agg_kernel(x_hbm, src_hbm, dst_hbm, out_hbm,
                   sidx, didx0, didx1, rows0, rows1, acc,
                   ga, gb, sa, sb, ia, ib):
        c = lax.axis_index("c")
        s = lax.axis_index("s")
        wid = s * NC + c
        rc = jnp.minimum(CH, CHUNKS_REAL - wid * CH)

        pltpu.async_copy(src_hbm.at[pl.ds(wid * CH, CH)], sidx, ia)
        pltpu.async_copy(dst_hbm.at[pl.ds(wid * CH, QCH)], didx0, ib)

        r0 = s * ROWS_PER_SUB

        @pl.when(s < NS - 1)
        def _():
            pltpu.sync_copy(x_hbm.at[pl.ds(r0, ROWS_PER_SUB)],
                            acc.at[pl.ds(r0, ROWS_PER_SUB)])

        @pl.when(s == NS - 1)
        def _():
            pltpu.sync_copy(x_hbm.at[pl.ds((NS - 1) * ROWS_PER_SUB, ROWS_LAST)],
                            acc.at[pl.ds((NS - 1) * ROWS_PER_SUB, ROWS_LAST)])

        pltpu.make_async_copy(
            src_hbm.at[pl.ds(wid * CH, CH)], sidx, ia).wait()
        pltpu.make_async_copy(
            dst_hbm.at[pl.ds(wid * CH, QCH)], didx0, ib).wait()

        bufs = (rows0, rows1)
        gsems = (ga, gb)
        ssems = (sa, sb)
        dbufs = (didx0, didx1)

        def gather(ch, buf, sem):
            pltpu.async_copy(x_hbm.at[sidx.at[ch]], buf, sem)

        def gather_wait(ch, buf, sem):
            pltpu.make_async_copy(x_hbm.at[sidx.at[ch]], buf, sem).wait()

        def scatter_wait(buf, sem):
            pltpu.make_async_copy(buf, acc.at[didx0.at[0]], sem).wait()

        gather(0, rows0, ga)
        plsc.subcore_barrier()

        for q in range(5):
            base = q * QCH
            cur = dbufs[q % 2]
            nxt = dbufs[1 - q % 2]

            if q >= 1:
                pltpu.make_async_copy(
                    dst_hbm.at[pl.ds(wid * CH + base, QCH)], cur, ia).wait()

                @pl.when(rc > base)
                def _():
                    scatter_wait(rows1, sb)

                @pl.when(jnp.logical_and(rc > base - QCH, rc <= base))
                def _():
                    scatter_wait(rows0, sa)
                    scatter_wait(rows1, sb)

            if q < 4:
                pltpu.async_copy(
                    dst_hbm.at[pl.ds(wid * CH + base + QCH, QCH)], nxt, ia)

            def body(j, carry, base=base, cur=cur):
                for b in range(2):
                    loc = j * 2 + b
                    ch = base + loc
                    buf, gsem, ssem = bufs[b], gsems[b], ssems[b]
                    nbuf, ngsem, nssem = bufs[1 - b], gsems[1 - b], ssems[1 - b]

                    @pl.when(ch < rc)
                    def _():
                        @pl.when(ch + 1 < rc)
                        def _():
                            @pl.when(loc >= 1)
                            def _():
                                scatter_wait(nbuf, nssem)

                            gather(ch + 1, nbuf, ngsem)

                        gather_wait(ch, buf, gsem)
                        pltpu.async_copy(
                            buf, acc.at[cur.at[loc]], ssem, add=True)

                return carry

            lax.fori_loop(0, QCH // 2, body, 0)

        @pl.when(rc > CH - QCH)
        def _():
            scatter_wait(rows0, sa)
            scatter_wait(rows1, sb)

        plsc.subcore_barrier()

        @pl.when(s < NS - 1)
        def _():
            pltpu.sync_copy(acc.at[pl.ds(r0, ROWS_PER_SUB)],
                            out_hbm.at[c, pl.ds(r0, ROWS_PER_SUB)])

        @pl.when(s == NS - 1)
        def _():
            pltpu.sync_copy(acc.at[pl.ds((NS - 1) * ROWS_PER_SUB, ROWS_LAST)],
                            out_hbm.at[c, pl.ds((NS - 1) * ROWS_PER_SUB, ROWS_LAST)])

    return agg_kernel(x, src2, dst2)


def _tc_mlp(x, p, epsm1, W1, W2, g, b):
    def body(x_ref, p_ref, e_ref, w1_ref, w2_ref, g_ref, b_ref, o_ref):
        h = x_ref[...] * e_ref[0, 0] + p_ref[0] + p_ref[1]
        h = lax.dot_general(h, w1_ref[...], (((1,), (1,)), ((), ())),
                            preferred_element_type=jnp.float32)
        h = jnp.maximum(h, 0.0)
        h = lax.dot_general(h, w2_ref[...], (((1,), (1,)), ((), ())),
                            preferred_element_type=jnp.float32)
        h = jnp.maximum(h, 0.0)
        mean = jnp.mean(h, axis=0, keepdims=True)
        var = jnp.mean(jnp.square(h - mean), axis=0, keepdims=True)
        o_ref[...] = (h - mean) * lax.rsqrt(var + 1e-5) * g_ref[...] + b_ref[...]

    return pl.pallas_call(
        body,
        out_shape=jax.ShapeDtypeStruct((N, D), jnp.float32),
    )(x, p, epsm1, W1, W2, g.reshape(1, D), b.reshape(1, D))


def kernel(x, adj, eps0, W1_0, W2_0, g0, b0, eps1, W1_1, W2_1, g1, b1):
    pad = E_PAD - E
    src2 = jnp.pad(adj[0], (0, pad)).reshape(NW * CH, K)
    dst2 = jnp.pad(adj[1], (0, pad)).reshape(NW * CH, K)
    p = _sc_agg(x, src2, dst2)
    h = _tc_mlp(x, p, (eps0 - 1.0).reshape(1, 1), W1_0, W2_0, g0, b0)
    p2 = _sc_agg(h, src2, dst2)
    return _tc_mlp(h, p2, (eps1 - 1.0).reshape(1, 1), W1_1, W2_1, g1, b1)
